# baseline (speedup 1.0000x reference)
import functools

import jax
import jax.numpy as jnp
from jax import lax
from jax.experimental import pallas as pl
from jax.experimental.pallas import tpu as pltpu
from jax.experimental.pallas import tpu_sc as plsc

_BM = 128


def _dispatch_body(r_ref, router_w_ref, sec_gate_w_ref,
                   top_vals_ref, meta_ref, slots_ref, fake_ref, eid_ref,
                   *, E, NSEC, T, NBLK):
    H = r_ref.shape[1]
    r = r_ref[...]
    logits = jax.lax.dot_general(
        router_w_ref[...], r, (((1,), (1,)), ((), ())),
        preferred_element_type=jnp.float32)
    sub = jax.lax.broadcasted_iota(jnp.int32, (E, T), 0)
    m1 = jnp.max(logits, axis=0, keepdims=True)
    i1 = jnp.min(jnp.where(logits == m1, sub, E), axis=0, keepdims=True)
    masked = jnp.where(sub == i1, -jnp.inf, logits)
    m2 = jnp.max(masked, axis=0, keepdims=True)
    i2 = jnp.min(jnp.where(masked == m2, sub, E), axis=0, keepdims=True)
    t = jnp.exp(m2 - m1)
    s = 1.0 / (1.0 + t)
    w1 = s
    w2 = t * s
    top_vals_ref[...] = jnp.transpose(jnp.concatenate([m1, m2], axis=0))

    slog = jax.lax.dot_general(
        sec_gate_w_ref[...].reshape(E * NSEC, H), r,
        (((1,), (1,)), ((), ())), preferred_element_type=jnp.float32)
    maskall = (slog > 0.0).astype(jnp.float32)

    c0 = (sub == i1).astype(jnp.float32)
    c1 = (sub == i2).astype(jnp.float32)
    cfull = jnp.concatenate([c0, c1], axis=1)
    counts = jnp.sum(cfull, axis=1, keepdims=True)

    d = jnp.transpose(cfull)
    nb = (2 * T) // 256
    si = jax.lax.broadcasted_iota(jnp.int32, (256, 256), 0)
    sj = jax.lax.broadcasted_iota(jnp.int32, (256, 256), 1)
    tril = (sj <= si).astype(jnp.float32)
    running = jnp.zeros((1, E), jnp.float32)
    excl_blocks = []
    for b in range(nb):
        db = d[b * 256:(b + 1) * 256, :]
        incl = jax.lax.dot_general(tril, db, (((1,), (0,)), ((), ())),
                                   preferred_element_type=jnp.float32)
        excl_blocks.append(incl - db + running)
        running = running + incl[255:256, :]
    rank = jnp.concatenate(excl_blocks, axis=0)

    counts_pad = jnp.floor((counts + (_BM - 1.0)) / _BM) * _BM
    stril = (jax.lax.broadcasted_iota(jnp.int32, (E, E), 1)
             < jax.lax.broadcasted_iota(jnp.int32, (E, E), 0)).astype(jnp.float32)
    base = jax.lax.dot_general(stril, counts_pad, (((1,), (0,)), ((), ())),
                               preferred_element_type=jnp.float32)
    base_row = jnp.transpose(base)
    rank_sel = jnp.sum(rank * d, axis=1, keepdims=True)
    base_sel = jnp.sum(d * base_row, axis=1, keepdims=True)
    slots_ref[...] = (base_sel + rank_sel).astype(jnp.int32)

    trash = float(2 * T + E * _BM)
    jf = jax.lax.broadcasted_iota(jnp.int32, (E, _BM), 1).astype(jnp.float32)
    ef = jax.lax.broadcasted_iota(jnp.int32, (E, _BM), 0).astype(jnp.float32)
    pad_amt = counts_pad - counts
    fake = jnp.where(jf < pad_amt, base + counts + jf,
                     trash + ef * _BM + jf)
    fake_ref[...] = fake.astype(jnp.int32)

    bstart = (jax.lax.broadcasted_iota(jnp.int32, (NBLK, E), 0)
              * _BM).astype(jnp.float32)
    eid = jnp.sum((bstart >= base_row).astype(jnp.int32), axis=1,
                  keepdims=True) - 1
    eid_ref[...] = eid

    ti = jax.lax.broadcasted_iota(jnp.int32, (1, T), 1).astype(jnp.float32)
    tok_row = jnp.concatenate([ti, ti], axis=1)
    w_row = jnp.concatenate([w1, w2], axis=1)
    mk = []
    for k, ck in ((0, c0), (1, c1)):
        acc = jnp.zeros((NSEC, T), jnp.float32)
        for e in range(E):
            acc = acc + maskall[e * NSEC:(e + 1) * NSEC, :] * ck[e:e + 1, :]
        mk.append(acc)
    mask_pair = jnp.concatenate(mk, axis=1)
    zpad = jnp.zeros((128 - 2 - NSEC, 2 * T), jnp.float32)
    meta_t = jnp.concatenate([tok_row, w_row, mask_pair, zpad], axis=0)
    meta_ref[...] = jnp.transpose(meta_t)


def _sc_mesh():
    return plsc.VectorSubcoreMesh(core_axis_name="c", subcore_axis_name="s")


def _wid():
    info = plsc.get_sparse_core_info()
    return lax.axis_index("s") * info.num_cores + lax.axis_index("c")


def _make_dispatch(NP, NF, P, T, H):
    nw = 32
    npw = NP // nw
    nfw = NF // nw

    @functools.partial(
        pl.kernel, mesh=_sc_mesh(),
        out_type=[
            jax.ShapeDtypeStruct((P, 128), jnp.float32),
            jax.ShapeDtypeStruct((P, H), jnp.float32),
        ],
        scratch_types=[
            pltpu.VMEM((npw,), jnp.int32),
            pltpu.VMEM((npw, 128), jnp.float32),
            pltpu.VMEM((nfw,), jnp.int32),
            pltpu.VMEM((nfw, 128), jnp.float32),
            pltpu.VMEM((npw, H), jnp.float32),
            pltpu.SemaphoreType.DMA,
            pltpu.SemaphoreType.DMA,
            pltpu.SemaphoreType.DMA,
        ],
    )
    def dispatch_k(meta_hbm, slots_hbm, fake_hbm, x_hbm, out_hbm, xs_hbm,
                   idx_v, meta_v, fidx_v, z_v, xrows_v, sem1, sem2, sem3):
        w = _wid()
        pltpu.sync_copy(slots_hbm.at[pl.ds(w * npw, npw)], idx_v)
        pltpu.sync_copy(meta_hbm.at[pl.ds(w * npw, npw)], meta_v)
        tok0 = lax.rem(w * npw, T)
        pltpu.sync_copy(x_hbm.at[pl.ds(tok0, npw)], xrows_v)
        for i in range(nfw):
            for j in range(8):
                z_v[i, pl.ds(j * 16, 16)] = jnp.zeros((16,), jnp.float32)
        pltpu.async_copy(meta_v, out_hbm.at[idx_v], sem1).wait()
        pltpu.async_copy(xrows_v, xs_hbm.at[idx_v], sem3).wait()
        pltpu.sync_copy(fake_hbm.at[pl.ds(w * nfw, nfw)], fidx_v)
        pltpu.async_copy(z_v, out_hbm.at[fidx_v], sem2).wait()

    return dispatch_k


def _make_combine(P, T, H):
    nw = 32
    tpw = T // nw

    @functools.partial(
        pl.kernel, mesh=_sc_mesh(),
        out_type=jax.ShapeDtypeStruct((T, H), jnp.float32),
        scratch_types=[
            pltpu.VMEM((tpw,), jnp.int32),
            pltpu.VMEM((tpw,), jnp.int32),
            pltpu.VMEM((tpw, H), jnp.float32),
            pltpu.VMEM((tpw, H), jnp.float32),
            pltpu.SemaphoreType.DMA,
            pltpu.SemaphoreType.DMA,
        ],
    )
    def combine_k(ys_hbm, s0_hbm, s1_hbm, out_hbm,
                  i0_v, i1_v, a_v, b_v, sem1, sem2):
        w = _wid()
        pltpu.sync_copy(s0_hbm.at[pl.ds(w * tpw, tpw)], i0_v)
        pltpu.sync_copy(s1_hbm.at[pl.ds(w * tpw, tpw)], i1_v)
        cp1 = pltpu.async_copy(ys_hbm.at[i0_v], a_v, sem1)
        cp2 = pltpu.async_copy(ys_hbm.at[i1_v], b_v, sem2)
        cp1.wait()
        cp2.wait()

        def _row(i, carry):
            for j in range(H // 16):
                sl = pl.ds(j * 16, 16)
                a_v[i, sl] = a_v[i, sl] + b_v[i, sl]
            return carry

        lax.fori_loop(0, tpw, _row, 0)
        pltpu.sync_copy(a_v, out_hbm.at[pl.ds(w * tpw, tpw)])

    return combine_k


def _ffn_body(eid_ref, xs_ref, meta_ref, up_ref, gate_ref, down_ref, ys_ref,
              *, NSEC, SEC):
    BM, H = xs_ref.shape
    FFN = up_ref.shape[1]
    x = xs_ref[...].astype(jnp.bfloat16)
    up = up_ref[0].astype(jnp.bfloat16)
    gate = gate_ref[0].astype(jnp.bfloat16)
    down = down_ref[0].astype(jnp.bfloat16)

    meta = meta_ref[...]
    wcol = meta[:, 1:2]
    smask = meta[:, 2:2 + NSEC] * wcol

    subj = jax.lax.broadcasted_iota(jnp.int32, (NSEC, FFN), 0)
    lanej = jax.lax.broadcasted_iota(jnp.int32, (NSEC, FFN), 1)
    expand = (lanej // SEC == subj).astype(jnp.float32)
    mask_full = jax.lax.dot_general(smask, expand, (((1,), (0,)), ((), ())),
                                    preferred_element_type=jnp.float32)

    u = jax.lax.dot_general(x, up, (((1,), (1,)), ((), ())),
                            preferred_element_type=jnp.float32)
    g = jax.lax.dot_general(x, gate, (((1,), (1,)), ((), ())),
                            preferred_element_type=jnp.float32)
    h = (u * mask_full * jnp.maximum(g, 0.0)).astype(jnp.bfloat16)
    ys_ref[...] = jax.lax.dot_general(h, down, (((1,), (1,)), ((), ())),
                                      preferred_element_type=jnp.float32)


def kernel(router_input, hidden_states, router_w, sec_gate_w, up_w, gate_w, down_w):
    B, S, H = hidden_states.shape
    T = B * S
    E, NSEC, _ = sec_gate_w.shape
    FFN = up_w.shape[1]
    SEC = FFN // NSEC
    NP = 2 * T
    NF = E * _BM
    P = NP + 2 * NF
    NBLK = P // _BM
    x = hidden_states.reshape(T, H)
    r = router_input.reshape(T, H)

    top_vals, meta, slots, fake, eid = pl.pallas_call(
        functools.partial(_dispatch_body, E=E, NSEC=NSEC, T=T, NBLK=NBLK),
        grid=(1,),
        in_specs=[
            pl.BlockSpec((T, H), lambda i: (0, 0)),
            pl.BlockSpec((E, H), lambda i: (0, 0)),
            pl.BlockSpec((E, NSEC, H), lambda i: (0, 0, 0)),
        ],
        out_specs=[
            pl.BlockSpec((T, 2), lambda i: (0, 0)),
            pl.BlockSpec((NP, 128), lambda i: (0, 0)),
            pl.BlockSpec((NP, 1), lambda i: (0, 0)),
            pl.BlockSpec((E, _BM), lambda i: (0, 0)),
            pl.BlockSpec((NBLK, 1), lambda i: (0, 0)),
        ],
        out_shape=[
            jax.ShapeDtypeStruct((T, 2), jnp.float32),
            jax.ShapeDtypeStruct((NP, 128), jnp.float32),
            jax.ShapeDtypeStruct((NP, 1), jnp.int32),
            jax.ShapeDtypeStruct((E, _BM), jnp.int32),
            jax.ShapeDtypeStruct((NBLK, 1), jnp.int32),
        ],
    )(r, router_w, sec_gate_w)

    slots1d = slots.reshape(NP)
    meta_sorted, xs = _make_dispatch(NP, NF, P, T, H)(
        meta, slots1d, fake.reshape(NF), x)

    ys = pl.pallas_call(
        functools.partial(_ffn_body, NSEC=NSEC, SEC=SEC),
        grid_spec=pltpu.PrefetchScalarGridSpec(
            num_scalar_prefetch=1,
            grid=(NBLK,),
            in_specs=[
                pl.BlockSpec((_BM, H), lambda b, eid: (b, 0)),
                pl.BlockSpec((_BM, 128), lambda b, eid: (b, 0)),
                pl.BlockSpec((1, FFN, H), lambda b, eid: (eid[b], 0, 0)),
                pl.BlockSpec((1, FFN, H), lambda b, eid: (eid[b], 0, 0)),
                pl.BlockSpec((1, H, FFN), lambda b, eid: (eid[b], 0, 0)),
            ],
            out_specs=pl.BlockSpec((_BM, H), lambda b, eid: (b, 0)),
        ),
        out_shape=jax.ShapeDtypeStruct((P, H), jnp.float32),
    )(eid.reshape(NBLK), xs, meta_sorted, up_w, gate_w, down_w)

    out = _make_combine(P, T, H)(ys, slots1d[:T], slots1d[T:])
    return out.reshape(B, S, H), top_vals


# speedup vs baseline: 1.5275x; 1.5275x over previous
import functools

import jax
import jax.numpy as jnp
from jax.experimental import pallas as pl
from jax.experimental.pallas import tpu as pltpu

_TC = 2048


def _moe_body(r_ref, x_ref, router_w_ref, sec_gate_w_ref, up_ref, gate_ref,
              down_ref, out_ref, top_vals_ref, combine_ref, maskall_ref,
              acc_ref, *, E, NSEC, SEC):
    e = pl.program_id(0)
    T, H = x_ref.shape
    FFN = up_ref.shape[1]

    @pl.when(e == 0)
    def _router():
        r = r_ref[...]
        logits = jax.lax.dot_general(
            router_w_ref[...], r, (((1,), (1,)), ((), ())),
            preferred_element_type=jnp.float32)
        sub = jax.lax.broadcasted_iota(jnp.int32, (E, T), 0)
        m1 = jnp.max(logits, axis=0, keepdims=True)
        i1 = jnp.min(jnp.where(logits == m1, sub, E), axis=0, keepdims=True)
        masked = jnp.where(sub == i1, -jnp.inf, logits)
        m2 = jnp.max(masked, axis=0, keepdims=True)
        i2 = jnp.min(jnp.where(masked == m2, sub, E), axis=0, keepdims=True)
        t = jnp.exp(m2 - m1)
        s = 1.0 / (1.0 + t)
        combine_ref[...] = (jnp.where(sub == i1, s, 0.0)
                            + jnp.where(sub == i2, t * s, 0.0))
        top_vals_ref[...] = jnp.transpose(
            jnp.concatenate([m1, m2], axis=0))
        slog = jax.lax.dot_general(
            sec_gate_w_ref[...].reshape(E * NSEC, H), r,
            (((1,), (1,)), ((), ())), preferred_element_type=jnp.float32)
        maskall_ref[...] = (slog > 0.0).astype(jnp.float32)

    up = up_ref[0].astype(jnp.bfloat16)
    gate = gate_ref[0].astype(jnp.bfloat16)
    down = down_ref[0].astype(jnp.bfloat16)

    subj = jax.lax.broadcasted_iota(jnp.int32, (FFN, NSEC), 0)
    lanes = jax.lax.broadcasted_iota(jnp.int32, (FFN, NSEC), 1)
    expand = (subj // SEC == lanes).astype(jnp.bfloat16)

    w_row = combine_ref[pl.ds(e, 1), :]
    mask_e = maskall_ref[pl.ds(e * NSEC, NSEC), :]
    mask_w = (mask_e * w_row).astype(jnp.bfloat16)

    FH = FFN // 2
    for c in range(T // _TC):
        sl = slice(c * _TC, (c + 1) * _TC)
        x = x_ref[sl, :]
        y_parts = []
        for f in range(2):
            fs = slice(f * FH, (f + 1) * FH)
            mask_f = jax.lax.dot_general(
                expand[fs, :], mask_w[:, sl], (((1,), (0,)), ((), ())),
                preferred_element_type=jnp.float32)
            u = jax.lax.dot_general(up[fs, :], x, (((1,), (1,)), ((), ())),
                                    preferred_element_type=jnp.float32)
            g = jax.lax.dot_general(gate[fs, :], x, (((1,), (1,)), ((), ())),
                                    preferred_element_type=jnp.float32)
            h = (u * mask_f * jnp.maximum(g, 0.0)).astype(jnp.bfloat16)
            y_parts.append(jax.lax.dot_general(
                down[:, fs], h, (((1,), (0,)), ((), ())),
                preferred_element_type=jnp.float32))
        y = y_parts[0] + y_parts[1]

        @pl.when(e == 0)
        def _init():
            acc_ref[:, sl] = y

        @pl.when(e != 0)
        def _acc():
            acc_ref[:, sl] = acc_ref[:, sl] + y

    @pl.when(e == E - 1)
    def _flush():
        out_ref[...] = jnp.transpose(acc_ref[...])


def kernel(router_input, hidden_states, router_w, sec_gate_w, up_w, gate_w, down_w):
    B, S, H = hidden_states.shape
    T = B * S
    E, NSEC, _ = sec_gate_w.shape
    FFN = up_w.shape[1]
    SEC = FFN // NSEC
    x = hidden_states.reshape(T, H)
    r = router_input.reshape(T, H)

    out, top_vals = pl.pallas_call(
        functools.partial(_moe_body, E=E, NSEC=NSEC, SEC=SEC),
        grid=(E,),
        in_specs=[
            pl.BlockSpec((T, H), lambda e: (0, 0)),
            pl.BlockSpec((T, H), lambda e: (0, 0)),
            pl.BlockSpec((E, H), lambda e: (0, 0)),
            pl.BlockSpec((E, NSEC, H), lambda e: (0, 0, 0)),
            pl.BlockSpec((1, FFN, H), lambda e: (e, 0, 0)),
            pl.BlockSpec((1, FFN, H), lambda e: (e, 0, 0)),
            pl.BlockSpec((1, H, FFN), lambda e: (e, 0, 0)),
        ],
        out_specs=[
            pl.BlockSpec((T, H), lambda e: (0, 0)),
            pl.BlockSpec((T, 2), lambda e: (0, 0)),
        ],
        out_shape=[
            jax.ShapeDtypeStruct((T, H), jnp.float32),
            jax.ShapeDtypeStruct((T, 2), jnp.float32),
        ],
        scratch_shapes=[
            pltpu.VMEM((E, T), jnp.float32),
            pltpu.VMEM((E * NSEC, T), jnp.float32),
            pltpu.VMEM((H, T), jnp.float32),
        ],
    )(r, x, router_w, sec_gate_w, up_w, gate_w, down_w)

    return out.reshape(B, S, H), top_vals
